# baseline (speedup 1.0000x reference)
import functools

import jax
import jax.numpy as jnp
from jax.experimental import pallas as pl


def _tokens_fold_sum(g, fold):
    rows = g.shape[0]
    blk = rows // fold
    parts = [g[c * blk:(c + 1) * blk] for c in range(fold)]
    while len(parts) > 1:
        parts = [parts[i] + parts[i + 1] for i in range(0, len(parts), 2)]
    ones = jnp.ones((1, blk), jnp.bfloat16)
    return jnp.dot(ones, parts[0], preferred_element_type=jnp.float32)


def _stage1_kernel(x_ref, w1_ref, b1_ref, out_ref, *, seg: int):
    x = x_ref[...].astype(jnp.bfloat16)
    fft = w1_ref.shape[1]
    cw = 256
    for c in range(fft // cw):
        sl = slice(c * cw, (c + 1) * cw)
        w = w1_ref[:, sl].astype(jnp.bfloat16)
        h = jnp.dot(x, w, preferred_element_type=jnp.float32)
        hb = (h + b1_ref[:, sl]).astype(jnp.bfloat16)
        g = jax.nn.gelu(hb)
        out_ref[0, 0, sl] = _tokens_fold_sum(g[:seg], 16)[0]
        out_ref[1, 0, sl] = _tokens_fold_sum(g[seg:], 16)[0]


def _stage2_kernel(p_ref, s_ref, w2_ref, b2_ref, out_ref):
    k = pl.program_id(0)
    inv_len = 1.0 / s_ref[:, 1:2].astype(jnp.float32)
    scaled = (p_ref[...] * inv_len).astype(jnp.bfloat16)
    w = w2_ref[...].astype(jnp.bfloat16)
    part = jnp.dot(scaled, w, preferred_element_type=jnp.float32)

    @pl.when(k == 0)
    def _init():
        out_ref[...] = b2_ref[...] + part

    @pl.when(k != 0)
    def _acc():
        out_ref[...] += part


def kernel(slices, inputs, W1, b1, W2, b2):
    b = slices.shape[0]
    tot, d = inputs.shape
    seg = tot // b
    ff = W1.shape[1]

    ff_tile = 1024
    nj = ff // ff_tile
    npair = b // 2
    b1r = b1.reshape(1, ff)
    b2r = b2.reshape(1, d)

    pooled = pl.pallas_call(
        functools.partial(_stage1_kernel, seg=seg),
        grid=(nj, npair),
        in_specs=[
            pl.BlockSpec((2 * seg, d), lambda j, p: (p, 0)),
            pl.BlockSpec((d, ff_tile), lambda j, p: (0, j)),
            pl.BlockSpec((1, ff_tile), lambda j, p: (0, j)),
        ],
        out_specs=pl.BlockSpec((2, 1, ff_tile), lambda j, p: (p, 0, j)),
        out_shape=jax.ShapeDtypeStruct((b, 1, ff), jnp.float32),
    )(inputs, W1, b1r)
    pooled = pooled.reshape(b, ff)

    k_tile = 2048
    nk = ff // k_tile
    out = pl.pallas_call(
        _stage2_kernel,
        grid=(nk,),
        in_specs=[
            pl.BlockSpec((b, k_tile), lambda k: (0, k)),
            pl.BlockSpec((b, 2), lambda k: (0, 0)),
            pl.BlockSpec((k_tile, d), lambda k: (k, 0)),
            pl.BlockSpec((1, d), lambda k: (0, 0)),
        ],
        out_specs=pl.BlockSpec((b, d), lambda k: (0, 0)),
        out_shape=jax.ShapeDtypeStruct((b, d), jnp.float32),
    )(pooled, slices, W2, b2r)
    return out


# speedup vs baseline: 1.6718x; 1.6718x over previous
import functools

import jax
import jax.numpy as jnp
from jax.experimental import pallas as pl
from jax.experimental.pallas import tpu as pltpu


def _tokens_fold_sum(g, fold):
    rows = g.shape[0]
    blk = rows // fold
    parts = [g[c * blk:(c + 1) * blk] for c in range(fold)]
    while len(parts) > 1:
        parts = [parts[i] + parts[i + 1] for i in range(0, len(parts), 2)]
    ones = jnp.ones((1, blk), jnp.bfloat16)
    return jnp.dot(ones, parts[0], preferred_element_type=jnp.float32)


def _stage1_kernel(x_ref, w1_ref, b1_ref, out_ref, *, seg: int):
    x = x_ref[...].astype(jnp.bfloat16)
    fft = w1_ref.shape[1]
    cw = 512
    for c in range(fft // cw):
        sl = slice(c * cw, (c + 1) * cw)
        w = w1_ref[:, sl].astype(jnp.bfloat16)
        h = jnp.dot(x, w, preferred_element_type=jnp.float32)
        hb = (h + b1_ref[:, sl]).astype(jnp.bfloat16)
        g = jax.nn.gelu(hb)
        out_ref[0, 0, sl] = _tokens_fold_sum(g[:seg], 16)[0]
        out_ref[1, 0, sl] = _tokens_fold_sum(g[seg:], 16)[0]


def _stage2_kernel(p_ref, s_ref, w2_ref, b2_ref, out_ref):
    k = pl.program_id(0)
    inv_len = 1.0 / s_ref[:, 1:2].astype(jnp.float32)
    scaled = (p_ref[...] * inv_len).astype(jnp.bfloat16)
    w = w2_ref[...].astype(jnp.bfloat16)
    part = jnp.dot(scaled, w, preferred_element_type=jnp.float32)

    @pl.when(k == 0)
    def _init():
        out_ref[...] = b2_ref[...] + part

    @pl.when(k != 0)
    def _acc():
        out_ref[...] += part


def kernel(slices, inputs, W1, b1, W2, b2):
    b = slices.shape[0]
    tot, d = inputs.shape
    seg = tot // b
    ff = W1.shape[1]

    ff_tile = 1024
    nj = ff // ff_tile
    npair = b // 2
    b1r = b1.reshape(1, ff)
    b2r = b2.reshape(1, d)

    pooled = pl.pallas_call(
        functools.partial(_stage1_kernel, seg=seg),
        grid=(nj, npair),
        in_specs=[
            pl.BlockSpec((2 * seg, d), lambda j, p: (p, 0)),
            pl.BlockSpec((d, ff_tile), lambda j, p: (0, j)),
            pl.BlockSpec((1, ff_tile), lambda j, p: (0, j)),
        ],
        out_specs=pl.BlockSpec((2, 1, ff_tile), lambda j, p: (p, 0, j)),
        out_shape=jax.ShapeDtypeStruct((b, 1, ff), jnp.float32),
        compiler_params=pltpu.CompilerParams(vmem_limit_bytes=62 * 1024 * 1024),
    )(inputs, W1, b1r)
    pooled = pooled.reshape(b, ff)

    k_tile = 2048
    nk = ff // k_tile
    out = pl.pallas_call(
        _stage2_kernel,
        grid=(nk,),
        in_specs=[
            pl.BlockSpec((b, k_tile), lambda k: (0, k)),
            pl.BlockSpec((b, 2), lambda k: (0, 0)),
            pl.BlockSpec((k_tile, d), lambda k: (k, 0)),
            pl.BlockSpec((1, d), lambda k: (0, 0)),
        ],
        out_specs=pl.BlockSpec((b, d), lambda k: (0, 0)),
        out_shape=jax.ShapeDtypeStruct((b, d), jnp.float32),
    )(pooled, slices, W2, b2r)
    return out


# speedup vs baseline: 1.9081x; 1.1413x over previous
import jax
import jax.numpy as jnp
from jax.experimental import pallas as pl


def _stage1_kernel(x_ref, w1_ref, b1_ref, out_ref):
    seg = x_ref.shape[0]
    x = x_ref[...].astype(jnp.bfloat16)
    w = w1_ref[...].astype(jnp.bfloat16)
    h = jnp.dot(x, w, preferred_element_type=jnp.float32)
    hb = (h + b1_ref[...]).astype(jnp.bfloat16)
    g = jax.nn.gelu(hb)
    fold = 16
    blk = seg // fold
    parts = [g[c * blk:(c + 1) * blk] for c in range(fold)]
    while len(parts) > 1:
        parts = [parts[i] + parts[i + 1] for i in range(0, len(parts), 2)]
    ones = jnp.ones((1, blk), jnp.bfloat16)
    out_ref[0, 0, :] = jnp.dot(ones, parts[0],
                               preferred_element_type=jnp.float32)[0]


def _stage2_kernel(p_ref, s_ref, w2_ref, b2_ref, out_ref):
    k = pl.program_id(0)
    inv_len = 1.0 / s_ref[:, 1:2].astype(jnp.float32)
    scaled = (p_ref[...] * inv_len).astype(jnp.bfloat16)
    w = w2_ref[...].astype(jnp.bfloat16)
    part = jnp.dot(scaled, w, preferred_element_type=jnp.float32)

    @pl.when(k == 0)
    def _init():
        out_ref[...] = b2_ref[...] + part

    @pl.when(k != 0)
    def _acc():
        out_ref[...] += part


def kernel(slices, inputs, W1, b1, W2, b2):
    b = slices.shape[0]
    tot, d = inputs.shape
    seg = tot // b
    ff = W1.shape[1]

    ff_tile = 2048
    nj = ff // ff_tile
    b1r = b1.reshape(1, ff)
    b2r = b2.reshape(1, d)

    pooled = pl.pallas_call(
        _stage1_kernel,
        grid=(nj, b),
        in_specs=[
            pl.BlockSpec((seg, d), lambda j, i: (i, 0)),
            pl.BlockSpec((d, ff_tile), lambda j, i: (0, j)),
            pl.BlockSpec((1, ff_tile), lambda j, i: (0, j)),
        ],
        out_specs=pl.BlockSpec((1, 1, ff_tile), lambda j, i: (i, 0, j)),
        out_shape=jax.ShapeDtypeStruct((b, 1, ff), jnp.float32),
    )(inputs, W1, b1r)
    pooled = pooled.reshape(b, ff)

    k_tile = 2048
    nk = ff // k_tile
    out = pl.pallas_call(
        _stage2_kernel,
        grid=(nk,),
        in_specs=[
            pl.BlockSpec((b, k_tile), lambda k: (0, k)),
            pl.BlockSpec((b, 2), lambda k: (0, 0)),
            pl.BlockSpec((k_tile, d), lambda k: (k, 0)),
            pl.BlockSpec((1, d), lambda k: (0, 0)),
        ],
        out_specs=pl.BlockSpec((b, d), lambda k: (0, 0)),
        out_shape=jax.ShapeDtypeStruct((b, d), jnp.float32),
    )(pooled, slices, W2, b2r)
    return out
